# initial kernel scaffold (unmeasured)
import jax
import jax.numpy as jnp
from jax import lax
from jax.experimental import pallas as pl
from jax.experimental.pallas import tpu as pltpu

T = 512
D = 1024
V_HALF = 8192
V = 2 * V_HALF
ROW_CHUNK = 128


def kernel(x, W):
    def body(x_ref, w_ref, o_ref, send_sem, recv_sem):
        my_x = lax.axis_index("x")
        my_y = lax.axis_index("y")
        my_z = lax.axis_index("z")
        nbr = (1 - my_x, my_y, my_z)

        barrier_sem = pltpu.get_barrier_semaphore()
        pl.semaphore_signal(
            barrier_sem, inc=1, device_id=nbr,
            device_id_type=pl.DeviceIdType.MESH,
        )
        pl.semaphore_wait(barrier_sem, 1)

        my_off = my_x * V_HALF
        o_ref[:, pl.ds(my_off, V_HALF)] = jnp.dot(
            x_ref[:, :], w_ref[:, :], preferred_element_type=jnp.float32
        )

        rdma = pltpu.make_async_remote_copy(
            src_ref=o_ref.at[:, pl.ds(my_off, V_HALF)],
            dst_ref=o_ref.at[:, pl.ds(my_off, V_HALF)],
            send_sem=send_sem,
            recv_sem=recv_sem,
            device_id=nbr,
            device_id_type=pl.DeviceIdType.MESH,
        )
        rdma.start()
        rdma.wait()

        for r in range(T // ROW_CHUNK):
            rows = pl.ds(r * ROW_CHUNK, ROW_CHUNK)
            logits = o_ref[rows, :]
            m = jnp.max(logits, axis=-1, keepdims=True)
            e = jnp.exp(logits - m)
            o_ref[rows, :] = e / jnp.sum(e, axis=-1, keepdims=True)

    return pl.pallas_call(
        body,
        out_shape=jax.ShapeDtypeStruct((T, V), jnp.float32),
        in_specs=[
            pl.BlockSpec(memory_space=pltpu.VMEM),
            pl.BlockSpec(memory_space=pltpu.VMEM),
        ],
        out_specs=pl.BlockSpec(memory_space=pltpu.VMEM),
        scratch_shapes=[
            pltpu.SemaphoreType.DMA,
            pltpu.SemaphoreType.DMA,
        ],
        compiler_params=pltpu.CompilerParams(collective_id=0),
    )(x, W)


# baseline (device time: 231781 ns/iter reference)
import jax
import jax.numpy as jnp
from jax import lax
from jax.experimental import pallas as pl
from jax.experimental.pallas import tpu as pltpu

T = 512
D = 1024
V_HALF = 8192
V = 2 * V_HALF
CHUNK = 1024
N_CHUNKS = V_HALF // CHUNK
ROW_CHUNK = 64


def kernel(x, W):
    def body(x_ref, w_hbm, o_ref, w_buf, load_sems, send_sems, recv_sems):
        my_x = lax.axis_index("x")
        my_y = lax.axis_index("y")
        my_z = lax.axis_index("z")
        nbr = (1 - my_x, my_y, my_z)

        barrier_sem = pltpu.get_barrier_semaphore()
        pl.semaphore_signal(
            barrier_sem, inc=1, device_id=nbr,
            device_id_type=pl.DeviceIdType.MESH,
        )
        pl.semaphore_wait(barrier_sem, 1)

        my_off = my_x * V_HALF

        def load(c):
            cp = pltpu.make_async_copy(
                w_hbm.at[:, pl.ds(c * CHUNK, CHUNK)],
                w_buf.at[c % 2],
                load_sems.at[c % 2],
            )
            cp.start()
            return cp

        rdmas = []
        nxt = load(0)
        for c in range(N_CHUNKS):
            cur, nxt = nxt, (load(c + 1) if c + 1 < N_CHUNKS else None)
            cur.wait()
            col = pl.ds(my_off + c * CHUNK, CHUNK)
            o_ref[:, col] = jnp.dot(
                x_ref[:, :], w_buf[c % 2], preferred_element_type=jnp.float32
            )
            rdma = pltpu.make_async_remote_copy(
                src_ref=o_ref.at[:, col],
                dst_ref=o_ref.at[:, col],
                send_sem=send_sems.at[c],
                recv_sem=recv_sems.at[c],
                device_id=nbr,
                device_id_type=pl.DeviceIdType.MESH,
            )
            rdma.start()
            rdmas.append(rdma)

        for rdma in rdmas:
            rdma.wait()

        def sm_body(r, carry):
            rows = pl.ds(r * ROW_CHUNK, ROW_CHUNK)
            logits = o_ref[rows, :]
            m = jnp.max(logits, axis=-1, keepdims=True)
            e = jnp.exp(logits - m)
            o_ref[rows, :] = e / jnp.sum(e, axis=-1, keepdims=True)
            return carry

        lax.fori_loop(0, T // ROW_CHUNK, sm_body, 0)

    return pl.pallas_call(
        body,
        out_shape=jax.ShapeDtypeStruct((T, V), jnp.float32),
        in_specs=[
            pl.BlockSpec(memory_space=pltpu.VMEM),
            pl.BlockSpec(memory_space=pl.ANY),
        ],
        out_specs=pl.BlockSpec(memory_space=pltpu.VMEM),
        scratch_shapes=[
            pltpu.VMEM((2, D, CHUNK), jnp.float32),
            pltpu.SemaphoreType.DMA((2,)),
            pltpu.SemaphoreType.DMA((N_CHUNKS,)),
            pltpu.SemaphoreType.DMA((N_CHUNKS,)),
        ],
        compiler_params=pltpu.CompilerParams(
            collective_id=0, vmem_limit_bytes=60 * 1024 * 1024
        ),
    )(x, W)
